# Initial kernel scaffold; baseline (speedup 1.0000x reference)
import jax
import jax.numpy as jnp
from jax.experimental import pallas as pl


def kernel(inputs, outputs, labels):
    raise NotImplementedError("write your pallas kernel here")


# speedup vs baseline: 1.1740x; 1.1740x over previous
import functools

import jax
import jax.numpy as jnp
from jax import lax
from jax.experimental import pallas as pl
from jax.experimental.pallas import tpu as pltpu
from jax.experimental.pallas import tpu_sc as plsc

B = 16384
C = 1000
SMOOTH = 0.1
EPS = 1e-12

NW = 32
PER_W = B // NW
LANES = 16
CHUNK = 128
NCHUNK = PER_W // CHUNK

ROWS_PER_STEP = 256
GRID = B // ROWS_PER_STEP


def _sc_gather(out_flat, labels):
    mesh = plsc.VectorSubcoreMesh(core_axis_name="c", subcore_axis_name="s")

    @functools.partial(
        pl.kernel,
        mesh=mesh,
        out_type=jax.ShapeDtypeStruct((B,), jnp.float32),
        scratch_types=[
            pltpu.VMEM((PER_W,), jnp.int32),
            pltpu.VMEM((NCHUNK, CHUNK), jnp.int32),
            pltpu.VMEM((PER_W,), jnp.float32),
            pltpu.SemaphoreType.DMA,
        ],
    )
    def k(table_hbm, labels_hbm, g_hbm, lbl_v, idx_v, g_v, sem):
        wid = lax.axis_index("s") * 2 + lax.axis_index("c")
        base = wid * PER_W
        pltpu.sync_copy(labels_hbm.at[pl.ds(base, PER_W)], lbl_v)
        for k_ in range(PER_W // LANES):
            lbl = lbl_v[pl.ds(k_ * LANES, LANES)]
            rows = base + k_ * LANES + lax.iota(jnp.int32, LANES)
            idx_v[k_ * LANES // CHUNK, pl.ds((k_ * LANES) % CHUNK, LANES)] = (
                rows * C + lbl)
        copies = [
            pltpu.async_copy(table_hbm.at[idx_v.at[c]],
                             g_v.at[pl.ds(c * CHUNK, CHUNK)], sem)
            for c in range(NCHUNK)
        ]
        for cp in copies:
            cp.wait()
        pltpu.sync_copy(g_v, g_hbm.at[pl.ds(base, PER_W)])

    return k(out_flat, labels)


def _dense_body(x_ref, g_ref, o_ref, acc_ref):
    step = pl.program_id(0)

    @pl.when(step == 0)
    def _():
        g = jnp.clip(g_ref[...], EPS, 1.0 - EPS)
        acc_ref[0, 0] = (1.0 - SMOOTH) * jnp.sum(jnp.log(g) - jnp.log(1.0 - g))

    p = jnp.clip(x_ref[...], EPS, 1.0 - EPS)
    term = SMOOTH * jnp.log(p) + (1.0 - SMOOTH) * jnp.log(1.0 - p)
    acc_ref[0, 0] += jnp.sum(term)

    @pl.when(step == GRID - 1)
    def _():
        o_ref[0, 0] = -acc_ref[0, 0] * (1.0 / (B * C))


def kernel(inputs, outputs, labels):
    del inputs
    g = _sc_gather(outputs.reshape(-1), labels.astype(jnp.int32))
    loss = pl.pallas_call(
        _dense_body,
        grid=(GRID,),
        in_specs=[
            pl.BlockSpec((ROWS_PER_STEP, C), lambda i: (i, 0)),
            pl.BlockSpec((128, 128), lambda i: (0, 0)),
        ],
        out_specs=pl.BlockSpec((1, 1), lambda i: (0, 0),
                               memory_space=pltpu.SMEM),
        out_shape=jax.ShapeDtypeStruct((1, 1), jnp.float32),
        scratch_shapes=[pltpu.SMEM((1, 1), jnp.float32)],
    )(outputs, g.reshape(128, 128))
    return loss[0, 0]
